# baseline (speedup 1.0000x reference)
import functools

import jax
import jax.numpy as jnp
from jax import lax
from jax.experimental import pallas as pl
from jax.experimental.pallas import tpu as pltpu
from jax.experimental.pallas import tpu_sc as plsc

N = 10000
E = 320000
D = 128
HID = 128
OUT = 128

NC = 2
NS = 16
NW = NC * NS
CHUNK = 80
EPW = E // NW
NSLOT = EPW // CHUNK
NBUF = 3
LEAD = 2
NPAD = 10112
ROWS_PER_TILE = NPAD // NS


def _sc_body(efeat_hbm, dst3_hbm, zeros_hbm, out_hbm, eout_hbm, *refs):
    idx_all = refs[0]
    rows = refs[1:1 + NBUF]
    agg_sh = refs[1 + NBUF]
    isem = refs[2 + NBUF]
    sems = refs[3 + NBUF:3 + 2 * NBUF]
    wsems = refs[3 + 2 * NBUF:3 + 3 * NBUF]
    ssems = refs[3 + 3 * NBUF:3 + 4 * NBUF]

    cid = lax.axis_index("c")
    sid = lax.axis_index("s")
    wid = sid * NC + cid

    pltpu.async_copy(dst3_hbm.at[wid], idx_all, isem)

    r0 = sid * ROWS_PER_TILE
    pltpu.sync_copy(zeros_hbm, agg_sh.at[pl.ds(r0, ROWS_PER_TILE)])
    pltpu.make_async_copy(dst3_hbm.at[wid], idx_all, isem).wait()
    plsc.subcore_barrier()

    ebase = wid * EPW
    bufs = tuple((rows[b], sems[b], wsems[b], ssems[b]) for b in range(NBUF))

    def row_copy(s, rows_v, sem):
        off = pl.multiple_of(ebase + s * CHUNK, 8)
        return pltpu.make_async_copy(
            efeat_hbm.at[pl.ds(off, CHUNK)], rows_v, sem)

    def wb_copy(s, rows_v, wsem):
        off = pl.multiple_of(ebase + s * CHUNK, 8)
        return pltpu.make_async_copy(
            rows_v, eout_hbm.at[pl.ds(off, CHUNK)], wsem)

    def scat_wait(s, rows_v, ssem):
        pltpu.make_async_copy(rows_v, agg_sh.at[idx_all.at[s]], ssem).wait()

    def load(s, b):
        rows_v, sem, wsem, ssem = bufs[b]

        @pl.when(s < NSLOT)
        def _():
            @pl.when(s >= NBUF)
            def _():
                scat_wait(s - NBUF, rows_v, ssem)
                wb_copy(s - NBUF, rows_v, wsem).wait()

            row_copy(s, rows_v, sem).start()

    def process(s, b):
        rows_v, sem, wsem, ssem = bufs[b]

        @pl.when(s < NSLOT)
        def _():
            row_copy(s, rows_v, sem).wait()
            wb_copy(s, rows_v, wsem).start()
            pltpu.async_copy(rows_v, agg_sh.at[idx_all.at[s]], ssem, add=True)

    for b in range(LEAD):
        load(b, b)

    def ring_body(i, carry):
        for b in range(NBUF):
            s = NBUF * i + b
            load(s + LEAD, (b + LEAD) % NBUF)
            process(s, b)
        return carry

    lax.fori_loop(0, -(-NSLOT // NBUF), ring_body, 0)
    for s in range(NSLOT - NBUF, NSLOT):
        rows_v, sem, wsem, ssem = bufs[s % NBUF]
        scat_wait(s, rows_v, ssem)
        wb_copy(s, rows_v, wsem).wait()

    plsc.subcore_barrier()

    pltpu.sync_copy(agg_sh.at[pl.ds(r0, ROWS_PER_TILE)],
                    out_hbm.at[cid, pl.ds(r0, ROWS_PER_TILE)])


def _sc_segment_partials(efeat, dst3, zeros):
    mesh = plsc.VectorSubcoreMesh(core_axis_name="c", subcore_axis_name="s")
    return pl.kernel(
        _sc_body,
        mesh=mesh,
        out_type=[jax.ShapeDtypeStruct((NC, NPAD, D), jnp.float32),
                  jax.ShapeDtypeStruct((E, D), jnp.float32)],
        scratch_types=(
            [pltpu.VMEM((NSLOT, CHUNK), jnp.int32)]
            + [pltpu.VMEM((CHUNK, D), jnp.float32)] * NBUF
            + [pltpu.VMEM_SHARED((NPAD, D), jnp.float32)]
            + [pltpu.SemaphoreType.DMA] * (1 + 3 * NBUF)
        ),
    )(efeat, dst3, zeros)


CPBLK = 8000


def _copy_body(x_ref, o_ref):
    o_ref[...] = x_ref[...]


def _tc_copy(x):
    return pl.pallas_call(
        _copy_body,
        grid=(E // CPBLK,),
        in_specs=[pl.BlockSpec((CPBLK, D), lambda i: (i, 0))],
        out_specs=pl.BlockSpec((CPBLK, D), lambda i: (i, 0)),
        out_shape=jax.ShapeDtypeStruct((E, D), jnp.float32),
    )(x)


BLK = 2000


def _tc_body(p_ref, nf_ref, w1a_ref, w1b_ref, b1_ref, w2_ref, b2_ref,
             g_ref, b_ref, out_ref):
    agg = p_ref[0] + p_ref[1]
    nf = nf_ref[...]
    h = (
        jnp.dot(agg, w1a_ref[...], preferred_element_type=jnp.float32)
        + jnp.dot(nf, w1b_ref[...], preferred_element_type=jnp.float32)
        + b1_ref[...]
    )
    h = h * jax.nn.sigmoid(h)
    h = jnp.dot(h, w2_ref[...], preferred_element_type=jnp.float32) + b2_ref[...]
    mean = jnp.mean(h, axis=-1, keepdims=True)
    var = jnp.mean((h - mean) ** 2, axis=-1, keepdims=True)
    h = (h - mean) * lax.rsqrt(var + 1e-5) * g_ref[...] + b_ref[...]
    out_ref[...] = h + nf


def _tc_mlp(partials, nfeat, w1a, w1b, b1, w2t, b2, gamma, beta):
    grid = (N // BLK,)
    return pl.pallas_call(
        _tc_body,
        grid=grid,
        in_specs=[
            pl.BlockSpec((NC, BLK, D), lambda i: (0, i, 0)),
            pl.BlockSpec((BLK, D), lambda i: (i, 0)),
            pl.BlockSpec((D, HID), lambda i: (0, 0)),
            pl.BlockSpec((D, HID), lambda i: (0, 0)),
            pl.BlockSpec((1, HID), lambda i: (0, 0)),
            pl.BlockSpec((HID, OUT), lambda i: (0, 0)),
            pl.BlockSpec((1, OUT), lambda i: (0, 0)),
            pl.BlockSpec((1, OUT), lambda i: (0, 0)),
            pl.BlockSpec((1, OUT), lambda i: (0, 0)),
        ],
        out_specs=pl.BlockSpec((BLK, OUT), lambda i: (i, 0)),
        out_shape=jax.ShapeDtypeStruct((N, OUT), jnp.float32),
    )(partials, nfeat, w1a, w1b, b1, w2t, b2, gamma, beta)


def kernel(efeat, nfeat, edge_index, W1, b1, W2, b2, gamma, beta):
    dst3 = edge_index[1].reshape(NW, NSLOT, CHUNK)
    zeros = jnp.zeros((ROWS_PER_TILE, D), dtype=jnp.float32)
    partials, efeat_out = _sc_segment_partials(efeat, dst3, zeros)

    w1t = W1.T
    w1a = w1t[:D]
    w1b = w1t[D:]
    w2t = W2.T
    nfeat_new = _tc_mlp(
        partials, nfeat, w1a, w1b, b1.reshape(1, HID), w2t,
        b2.reshape(1, OUT), gamma.reshape(1, OUT), beta.reshape(1, OUT))
    return (efeat_out, nfeat_new)


# speedup vs baseline: 1.0782x; 1.0782x over previous
import functools

import jax
import jax.numpy as jnp
from jax import lax
from jax.experimental import pallas as pl
from jax.experimental.pallas import tpu as pltpu
from jax.experimental.pallas import tpu_sc as plsc

N = 10000
E = 320000
D = 128
HID = 128
OUT = 128

NC = 2
NS = 16
NW = NC * NS
CHUNK = 64
TOTAL_CHUNKS = E // CHUNK
NSLOT = -(-TOTAL_CHUNKS // NW)
NBUF = 6
LEAD = 2
NPAD = 10112
ROWS_PER_TILE = NPAD // NS


def _sc_body(efeat_hbm, eif_hbm, zeros_hbm, out_hbm, eout_hbm, *refs):
    idxs = refs[0:NBUF]
    rows = refs[NBUF:2 * NBUF]
    agg_sh = refs[2 * NBUF]
    sems = refs[2 * NBUF + 1:2 * NBUF + 1 + NBUF]
    wsems = refs[2 * NBUF + 1 + NBUF:2 * NBUF + 1 + 2 * NBUF]
    ssems = refs[2 * NBUF + 1 + 2 * NBUF:2 * NBUF + 1 + 3 * NBUF]

    cid = lax.axis_index("c")
    sid = lax.axis_index("s")
    wid = sid * NC + cid

    r0 = sid * ROWS_PER_TILE
    pltpu.sync_copy(zeros_hbm, agg_sh.at[pl.ds(r0, ROWS_PER_TILE)])
    plsc.subcore_barrier()

    bufs = tuple(
        (idxs[b], rows[b], sems[b], wsems[b], ssems[b]) for b in range(NBUF))

    def q_of(s):
        return s * NW + wid

    def e_off(s):
        return pl.multiple_of(q_of(s) * CHUNK, 8)

    def idx_copy(s, idx_v, sem):
        return pltpu.make_async_copy(
            eif_hbm.at[pl.ds(E + e_off(s), CHUNK)], idx_v, sem)

    def row_copy(s, rows_v, sem):
        return pltpu.make_async_copy(
            efeat_hbm.at[pl.ds(e_off(s), CHUNK)], rows_v, sem)

    def wb_copy(s, rows_v, wsem):
        return pltpu.make_async_copy(
            rows_v, eout_hbm.at[pl.ds(e_off(s), CHUNK)], wsem)

    def scat_wait(idx_v, rows_v, ssem):
        pltpu.make_async_copy(rows_v, agg_sh.at[idx_v], ssem).wait()

    def load(s, b):
        idx_v, rows_v, sem, wsem, ssem = bufs[b]

        @pl.when(q_of(s) < TOTAL_CHUNKS)
        def _():
            @pl.when((s >= NBUF) & (q_of(s - NBUF) < TOTAL_CHUNKS))
            def _():
                scat_wait(idx_v, rows_v, ssem)
                wb_copy(s - NBUF, rows_v, wsem).wait()

            idx_copy(s, idx_v, sem).start()
            row_copy(s, rows_v, sem).start()

    def process(s, b):
        idx_v, rows_v, sem, wsem, ssem = bufs[b]

        @pl.when(q_of(s) < TOTAL_CHUNKS)
        def _():
            idx_copy(s, idx_v, sem).wait()
            row_copy(s, rows_v, sem).wait()
            wb_copy(s, rows_v, wsem).start()
            pltpu.async_copy(rows_v, agg_sh.at[idx_v], ssem, add=True)

    for b in range(LEAD):
        load(b, b)

    def ring_body(i, carry):
        for b in range(NBUF):
            s = NBUF * i + b
            load(s + LEAD, (b + LEAD) % NBUF)
            process(s, b)
        return carry

    lax.fori_loop(0, -(-NSLOT // NBUF), ring_body, 0)
    for s in range(max(0, NSLOT - NBUF - 1), NSLOT):
        idx_v, rows_v, sem, wsem, ssem = bufs[s % NBUF]

        @pl.when((q_of(s) < TOTAL_CHUNKS)
                 & (q_of(s + NBUF) >= TOTAL_CHUNKS))
        def _():
            scat_wait(idx_v, rows_v, ssem)
            wb_copy(s, rows_v, wsem).wait()

    plsc.subcore_barrier()

    pltpu.sync_copy(agg_sh.at[pl.ds(r0, ROWS_PER_TILE)],
                    out_hbm.at[cid, pl.ds(r0, ROWS_PER_TILE)])


def _sc_segment_partials(efeat, eif, zeros):
    mesh = plsc.VectorSubcoreMesh(core_axis_name="c", subcore_axis_name="s")
    return pl.kernel(
        _sc_body,
        mesh=mesh,
        out_type=[jax.ShapeDtypeStruct((NC, NPAD, D), jnp.float32),
                  jax.ShapeDtypeStruct((E, D), jnp.float32)],
        scratch_types=(
            [pltpu.VMEM((CHUNK,), jnp.int32)] * NBUF
            + [pltpu.VMEM((CHUNK, D), jnp.float32)] * NBUF
            + [pltpu.VMEM_SHARED((NPAD, D), jnp.float32)]
            + [pltpu.SemaphoreType.DMA] * (3 * NBUF)
        ),
    )(efeat, eif, zeros)


CPBLK = 8000


def _copy_body(x_ref, o_ref):
    o_ref[...] = x_ref[...]


def _tc_copy(x):
    return pl.pallas_call(
        _copy_body,
        grid=(E // CPBLK,),
        in_specs=[pl.BlockSpec((CPBLK, D), lambda i: (i, 0))],
        out_specs=pl.BlockSpec((CPBLK, D), lambda i: (i, 0)),
        out_shape=jax.ShapeDtypeStruct((E, D), jnp.float32),
    )(x)


BLK = 2000


def _tc_body(p_ref, nf_ref, w1a_ref, w1b_ref, b1_ref, w2_ref, b2_ref,
             g_ref, b_ref, out_ref):
    agg = p_ref[0] + p_ref[1]
    nf = nf_ref[...]
    h = (
        jnp.dot(agg, w1a_ref[...], preferred_element_type=jnp.float32)
        + jnp.dot(nf, w1b_ref[...], preferred_element_type=jnp.float32)
        + b1_ref[...]
    )
    h = h * jax.nn.sigmoid(h)
    h = jnp.dot(h, w2_ref[...], preferred_element_type=jnp.float32) + b2_ref[...]
    mean = jnp.mean(h, axis=-1, keepdims=True)
    var = jnp.mean((h - mean) ** 2, axis=-1, keepdims=True)
    h = (h - mean) * lax.rsqrt(var + 1e-5) * g_ref[...] + b_ref[...]
    out_ref[...] = h + nf


def _tc_mlp(partials, nfeat, w1a, w1b, b1, w2t, b2, gamma, beta):
    grid = (N // BLK,)
    return pl.pallas_call(
        _tc_body,
        grid=grid,
        in_specs=[
            pl.BlockSpec((NC, BLK, D), lambda i: (0, i, 0)),
            pl.BlockSpec((BLK, D), lambda i: (i, 0)),
            pl.BlockSpec((D, HID), lambda i: (0, 0)),
            pl.BlockSpec((D, HID), lambda i: (0, 0)),
            pl.BlockSpec((1, HID), lambda i: (0, 0)),
            pl.BlockSpec((HID, OUT), lambda i: (0, 0)),
            pl.BlockSpec((1, OUT), lambda i: (0, 0)),
            pl.BlockSpec((1, OUT), lambda i: (0, 0)),
            pl.BlockSpec((1, OUT), lambda i: (0, 0)),
        ],
        out_specs=pl.BlockSpec((BLK, OUT), lambda i: (i, 0)),
        out_shape=jax.ShapeDtypeStruct((N, OUT), jnp.float32),
    )(partials, nfeat, w1a, w1b, b1, w2t, b2, gamma, beta)


def kernel(efeat, nfeat, edge_index, W1, b1, W2, b2, gamma, beta):
    eif = edge_index.reshape(-1)
    zeros = jnp.zeros((ROWS_PER_TILE, D), dtype=jnp.float32)
    partials, efeat_out = _sc_segment_partials(efeat, eif, zeros)

    w1t = W1.T
    w1a = w1t[:D]
    w1b = w1t[D:]
    w2t = W2.T
    nfeat_new = _tc_mlp(
        partials, nfeat, w1a, w1b, b1.reshape(1, HID), w2t,
        b2.reshape(1, OUT), gamma.reshape(1, OUT), beta.reshape(1, OUT))
    return (efeat_out, nfeat_new)
